# baseline (speedup 1.0000x reference)
import functools

import jax
import jax.numpy as jnp
from jax import lax
from jax.experimental import pallas as pl
from jax.experimental.pallas import tpu as pltpu
from jax.experimental.pallas import tpu_sc as plsc

VOCAB = 1000000
D_EMBED = 64
BATCH = 16384

_info = plsc.get_sparse_core_info()
_NC, _NS = _info.num_cores, _info.num_subcores
_NW = _NC * _NS
_B_PER_W = BATCH // _NW
_L = 16
_NG = _B_PER_W // _L
_RING = 8
_DB = D_EMBED // 8


def _build_gather():
    mesh = plsc.VectorSubcoreMesh(core_axis_name="c", subcore_axis_name="s")

    @functools.partial(
        pl.kernel,
        mesh=mesh,
        out_type=jax.ShapeDtypeStruct((D_EMBED, BATCH), jnp.float32),
        scratch_types=[
            pltpu.VMEM((_B_PER_W,), jnp.int32),
            pltpu.VMEM((_RING, _DB, 8, 128), jnp.float32),
            pltpu.VMEM((D_EMBED, _B_PER_W), jnp.float32),
            pltpu.SemaphoreType.DMA((_RING,)),
        ],
        compiler_params=pltpu.CompilerParams(
            use_tc_tiling_on_sc=True, needs_layout_passes=False
        ),
    )
    def gather_kernel(idx_hbm, table_hbm, out_hbm, idx_v, ring_v, h_v, sems):
        wid = lax.axis_index("s") * _NC + lax.axis_index("c")
        base = wid * _B_PER_W
        pltpu.sync_copy(idx_hbm.at[pl.ds(base, _B_PER_W)], idx_v)

        iota = lax.iota(jnp.int32, _L)
        dblk_lo = iota // 8
        dsub_lo = iota % 8

        def issue(j, cb):
            cba = pl.multiple_of(cb, 128)
            for db in range(_DB):
                pltpu.make_async_copy(
                    table_hbm.at[db, :, pl.ds(cba, 128)],
                    ring_v.at[j % _RING, db],
                    sems.at[j % _RING],
                ).start()

        def select(j, col, kl):
            pltpu.make_async_copy(
                table_hbm.at[:, :, pl.ds(0, 128)],
                ring_v.at[j % _RING],
                sems.at[j % _RING],
            ).wait()
            col16 = jnp.zeros((_L,), jnp.int32) + col
            k16 = jnp.zeros((_L,), jnp.int32) + kl
            ring16 = jnp.zeros((_L,), jnp.int32) + (j % _RING)
            for half in range(4):
                x = plsc.load_gather(
                    ring_v,
                    [ring16, dblk_lo + 2 * half, dsub_lo, col16],
                )
                plsc.store_scatter(h_v, [iota + half * _L, k16], x)
            return None

        def group(q, _):
            k0 = q * _L
            v16 = idx_v[pl.ds(k0, _L)]
            cb16 = v16 & jnp.int32(-128)
            vs16 = idx_v[pl.ds(jnp.maximum(k0 - _RING, 0), _L)]
            co16 = vs16 & jnp.int32(127)
            for j in range(_L):
                if j < _RING:
                    @pl.when(q > 0)
                    def _():
                        select(j, co16[j], k0 + j - _RING)
                else:
                    col = jnp.where(q > 0, co16[j], v16[j - _RING] & 127)
                    select(j, col, k0 + j - _RING)
                issue(j, cb16[j])
            return 0

        lax.fori_loop(0, _NG, group, 0, unroll=False)
        vlast = idx_v[pl.ds(_B_PER_W - _L, _L)]
        clast = vlast & jnp.int32(127)
        for j in range(_RING):
            select(j + _RING, clast[j + _RING], _B_PER_W - _RING + j)
        pltpu.sync_copy(h_v, out_hbm.at[:, pl.ds(base, _B_PER_W)])

    return gather_kernel


_gather = _build_gather()


def kernel(indices, W):
    idx = indices.astype(jnp.int32)
    table3 = jnp.reshape(jnp.transpose(W), (8, 8, VOCAB))
    out_t = _gather(idx, table3)
    return jnp.transpose(out_t)


# speedup vs baseline: 1.0094x; 1.0094x over previous
import functools

import jax
import jax.numpy as jnp
from jax import lax
from jax.experimental import pallas as pl
from jax.experimental.pallas import tpu as pltpu
from jax.experimental.pallas import tpu_sc as plsc

VOCAB = 1000000
D_EMBED = 64
BATCH = 16384

_info = plsc.get_sparse_core_info()
_NC, _NS = _info.num_cores, _info.num_subcores
_NW = _NC * _NS
_B_PER_W = BATCH // _NW
_L = 16
_NG = _B_PER_W // _L
_RING = 8
_DB = D_EMBED // 8


def _build_gather():
    mesh = plsc.VectorSubcoreMesh(core_axis_name="c", subcore_axis_name="s")

    @functools.partial(
        pl.kernel,
        mesh=mesh,
        out_type=jax.ShapeDtypeStruct((D_EMBED, BATCH), jnp.float32),
        scratch_types=[
            pltpu.VMEM((_B_PER_W,), jnp.int32),
            pltpu.VMEM((_RING, _DB, 8, 128), jnp.float32),
            pltpu.VMEM((D_EMBED, _B_PER_W), jnp.float32),
            pltpu.SemaphoreType.DMA((_RING,)),
        ],
        compiler_params=pltpu.CompilerParams(
            use_tc_tiling_on_sc=True, needs_layout_passes=False
        ),
    )
    def gather_kernel(idx_hbm, table_hbm, out_hbm, idx_v, ring_v, h_v, sems):
        wid = lax.axis_index("s") * _NC + lax.axis_index("c")
        base = wid * _B_PER_W
        pltpu.sync_copy(idx_hbm.at[pl.ds(base, _B_PER_W)], idx_v)

        iota = lax.iota(jnp.int32, _L)
        dblk_lo = iota // 8
        dsub_lo = iota % 8

        def issue(j, cb):
            pltpu.make_async_copy(
                table_hbm.at[:, :, pl.ds(pl.multiple_of(cb, 128), 128)],
                ring_v.at[j % _RING],
                sems.at[j % _RING],
            ).start()

        def select(j, col, kl):
            pltpu.make_async_copy(
                table_hbm.at[:, :, pl.ds(0, 128)],
                ring_v.at[j % _RING],
                sems.at[j % _RING],
            ).wait()
            col16 = jnp.zeros((_L,), jnp.int32) + col
            k16 = jnp.zeros((_L,), jnp.int32) + kl
            ring16 = jnp.zeros((_L,), jnp.int32) + (j % _RING)
            for half in range(4):
                x = plsc.load_gather(
                    ring_v,
                    [ring16, dblk_lo + 2 * half, dsub_lo, col16],
                )
                plsc.store_scatter(h_v, [iota + half * _L, k16], x)
            return None

        def group(q, _):
            k0 = q * _L
            v16 = idx_v[pl.ds(k0, _L)]
            cb16 = v16 & jnp.int32(-128)
            vs16 = idx_v[pl.ds(jnp.maximum(k0 - _RING, 0), _L)]
            co16 = vs16 & jnp.int32(127)
            for j in range(_L):
                if j < _RING:
                    @pl.when(q > 0)
                    def _():
                        select(j, co16[j], k0 + j - _RING)
                else:
                    col = jnp.where(q > 0, co16[j], v16[j - _RING] & 127)
                    select(j, col, k0 + j - _RING)
                issue(j, cb16[j])
            return 0

        lax.fori_loop(0, _NG, group, 0, unroll=False)
        vlast = idx_v[pl.ds(_B_PER_W - _L, _L)]
        clast = vlast & jnp.int32(127)
        for j in range(_RING):
            select(j + _RING, clast[j + _RING], _B_PER_W - _RING + j)
        pltpu.sync_copy(h_v, out_hbm.at[:, pl.ds(base, _B_PER_W)])

    return gather_kernel


_gather = _build_gather()


def kernel(indices, W):
    idx = indices.astype(jnp.int32)
    table3 = jnp.reshape(jnp.transpose(W), (8, 8, VOCAB))
    out_t = _gather(idx, table3)
    return jnp.transpose(out_t)
